# baseline (speedup 1.0000x reference)
import functools

import jax
import jax.numpy as jnp
from jax import lax
from jax.experimental import pallas as pl
from jax.experimental.pallas import tpu as pltpu
from jax.experimental.pallas import tpu_sc as plsc

B = 4
T = 4096
V = 8192
D = 1024
NC = 2
NS = 16
L = 16
NW = NC * NS
CH = 32
NJ = 8
NBUF = 3

_mesh = plsc.VectorSubcoreMesh(
    core_axis_name="c", subcore_axis_name="s", num_cores=NC, num_subcores=NS
)


@functools.partial(
    pl.kernel,
    out_type=jax.ShapeDtypeStruct((B * T, D), jnp.float32),
    mesh=_mesh,
    compiler_params=pltpu.CompilerParams(
        needs_layout_passes=False,
        disable_bounds_checks=True,
        disable_semaphore_checks=True,
        skip_device_barrier=True,
    ),
    scratch_types=[
        pltpu.VMEM((L,), jnp.int32),
        pltpu.VMEM((NBUF * B, CH), jnp.int32),
        [pltpu.VMEM((CH, D), jnp.float32)] * NBUF,
        pltpu.VMEM((L, D), jnp.float32),
        [pltpu.SemaphoreType.DMA] * NBUF,
        [pltpu.SemaphoreType.DMA] * (NBUF * B),
        [pltpu.SemaphoreType.DMA] * 2,
    ],
)
def _sc_gather(off_hbm, weight_hbm, out_hbm, off_v, idxw, bufs, bufp, sg, ss, sp):
    wid = lax.axis_index("s") * NC + lax.axis_index("c")

    pltpu.sync_copy(off_hbm, off_v.at[pl.ds(0, B)])
    off_vec = off_v[...]
    lane = lax.iota(jnp.int32, L)
    offs = [jnp.sum(jnp.where(lane == b, off_vec, 0)) for b in range(B)]
    o_min = jnp.min(jnp.where(lane < B, off_vec, V))
    o_max = jnp.max(jnp.where(lane < B, off_vec, 0))
    gstart = pl.multiple_of(o_min - o_min % 8, 8)
    gend = o_max + T
    ramp = lax.iota(jnp.int32, L)

    def chunk_conds(jj):
        c0 = gstart + (wid + jj * NW) * CH
        act = jnp.logical_and(c0 + CH <= V, c0 < gend)
        cov = [
            jnp.logical_and(act, jnp.logical_and(offs[b] <= c0, c0 + CH <= offs[b] + T))
            for b in range(B)
        ]
        return c0, act, cov

    is_patch = wid < 16
    pb = wid // 4
    e = wid % 4
    ps = (e % 2) * L + (e // 2) * (T - 2 * L)
    ob = jnp.sum(jnp.where(lane == pb, off_vec, 0))
    patch_src = jnp.full((L,), ob + ps, jnp.int32) + ramp
    patch_g = pltpu.make_async_copy(weight_hbm.at[patch_src], bufp, sp[0])
    patch_s = pltpu.make_async_copy(
        bufp, out_hbm.at[pl.ds(pl.multiple_of(pb * T + ps, 8), L)], sp[1]
    )
    pl.when(is_patch)(patch_g.start)

    gathers = [None] * NJ
    scatters = [[None] * B for _ in range(NJ)]

    LAG = 2

    for jj in range(NJ + LAG):
        p = jj % NBUF
        if jj < NJ:
            c0, act, cov = chunk_conds(jj)
            if jj >= NBUF:
                _, _, covp = chunk_conds(jj - NBUF)
                for b in range(B):
                    pl.when(covp[b])(scatters[jj - NBUF][b].wait)
            gathers[jj] = pltpu.make_async_copy(
                weight_hbm.at[pl.ds(pl.multiple_of(c0, 8), CH)], bufs[p], sg[p]
            )
            pl.when(act)(gathers[jj].start)
            for b in range(B):
                dest0 = b * T + (c0 - offs[b])
                for h in range(CH // L):
                    idxw[p * B + b, pl.ds(h * L, L)] = (
                        jnp.full((L,), dest0 + h * L, jnp.int32) + ramp
                    )
        if jj >= LAG:
            q = (jj - LAG) % NBUF
            _, actq, covq = chunk_conds(jj - LAG)
            pl.when(actq)(gathers[jj - LAG].wait)
            for b in range(B):
                scatters[jj - LAG][b] = pltpu.make_async_copy(
                    bufs[q], out_hbm.at[idxw.at[q * B + b]], ss[q * B + b]
                )
                pl.when(covq[b])(scatters[jj - LAG][b].start)
    @pl.when(is_patch)
    def _patch_scatter():
        patch_g.wait()
        patch_s.start()

    for jj in range(NJ - NBUF, NJ):
        _, _, cov = chunk_conds(jj)
        for b in range(B):
            pl.when(cov[b])(scatters[jj][b].wait)
    pl.when(is_patch)(patch_s.wait)


def kernel(length, offset, weight):
    del length
    out = _sc_gather(jnp.ravel(offset).astype(jnp.int32), weight)
    return out.reshape(B, T, D)


# speedup vs baseline: 1.0091x; 1.0091x over previous
import functools

import jax
import jax.numpy as jnp
from jax import lax
from jax.experimental import pallas as pl
from jax.experimental.pallas import tpu as pltpu
from jax.experimental.pallas import tpu_sc as plsc

B = 4
T = 4096
V = 8192
D = 1024
NC = 2
NS = 16
L = 16
NW = NC * NS
CH = 32
NJ = 8
NBUF = 3

_mesh = plsc.VectorSubcoreMesh(
    core_axis_name="c", subcore_axis_name="s", num_cores=NC, num_subcores=NS
)


@functools.partial(
    pl.kernel,
    out_type=jax.ShapeDtypeStruct((B * T, D), jnp.float32),
    mesh=_mesh,
    compiler_params=pltpu.CompilerParams(
        needs_layout_passes=False,
        disable_bounds_checks=True,
        disable_semaphore_checks=True,
        skip_device_barrier=True,
    ),
    scratch_types=[
        pltpu.VMEM((L,), jnp.int32),
        pltpu.VMEM((NBUF * B, CH), jnp.int32),
        [pltpu.VMEM((CH, D), jnp.float32)] * NBUF,
        pltpu.VMEM((L, D), jnp.float32),
        [pltpu.SemaphoreType.DMA] * NBUF,
        [pltpu.SemaphoreType.DMA] * (NBUF * B),
        [pltpu.SemaphoreType.DMA] * 2,
    ],
)
def _sc_gather(off_hbm, weight_hbm, out_hbm, off_v, idxw, bufs, bufp, sg, ss, sp):
    wid = lax.axis_index("s") * NC + lax.axis_index("c")

    pltpu.sync_copy(off_hbm, off_v.at[pl.ds(0, B)])
    off_vec = off_v[...]
    lane = lax.iota(jnp.int32, L)
    offs = [jnp.sum(jnp.where(lane == b, off_vec, 0)) for b in range(B)]
    o_min = jnp.min(jnp.where(lane < B, off_vec, V))
    o_max = jnp.max(jnp.where(lane < B, off_vec, 0))
    gstart = pl.multiple_of(o_min - o_min % 8, 8)
    gend = o_max + T
    ramp = lax.iota(jnp.int32, L)

    def chunk_conds(jj):
        c0 = gstart + (wid + jj * NW) * CH
        act = jnp.logical_and(c0 + CH <= V, c0 < gend)
        cov = [
            jnp.logical_and(act, jnp.logical_and(offs[b] <= c0, c0 + CH <= offs[b] + T))
            for b in range(B)
        ]
        return c0, act, cov

    is_patch = wid < 16
    pb = wid // 4
    e = wid % 4
    ps = (e % 2) * L + (e // 2) * (T - 2 * L)
    ob = jnp.sum(jnp.where(lane == pb, off_vec, 0))
    patch_src = jnp.full((L,), ob + ps, jnp.int32) + ramp
    patch_g = pltpu.make_async_copy(weight_hbm.at[patch_src], bufp, sp[0])
    patch_s = pltpu.make_async_copy(
        bufp, out_hbm.at[pl.ds(pl.multiple_of(pb * T + ps, 8), L)], sp[1]
    )
    pl.when(is_patch)(patch_g.start)

    gathers = [None] * NJ
    scatters = [[None] * B for _ in range(NJ)]

    LAG = 2

    for jj in range(NJ + LAG):
        p = jj % NBUF
        if jj < NJ:
            c0, act, cov = chunk_conds(jj)
            if jj >= NBUF:
                _, _, covp = chunk_conds(jj - NBUF)
                for b in range(B):
                    pl.when(covp[b])(scatters[jj - NBUF][b].wait)
            for b in range(B):
                dest0 = b * T + (c0 - offs[b])
                for h in range(CH // L):
                    idxw[p * B + b, pl.ds(h * L, L)] = (
                        jnp.full((L,), dest0 + h * L, jnp.int32) + ramp
                    )
            gathers[jj] = pltpu.make_async_copy(
                weight_hbm.at[pl.ds(pl.multiple_of(c0, 8), CH)], bufs[p], sg[p]
            )
            pl.when(act)(gathers[jj].start)
        if jj >= LAG:
            q = (jj - LAG) % NBUF
            _, actq, covq = chunk_conds(jj - LAG)
            pl.when(actq)(gathers[jj - LAG].wait)
            for b in range(B):
                scatters[jj - LAG][b] = pltpu.make_async_copy(
                    bufs[q], out_hbm.at[idxw.at[q * B + b]], ss[q * B + b]
                )
                pl.when(covq[b])(scatters[jj - LAG][b].start)
    @pl.when(is_patch)
    def _patch_scatter():
        patch_g.wait()
        patch_s.start()

    for jj in range(NJ - NBUF, NJ):
        _, _, cov = chunk_conds(jj)
        for b in range(B):
            pl.when(cov[b])(scatters[jj][b].wait)
    pl.when(is_patch)(patch_s.wait)


def kernel(length, offset, weight):
    del length
    out = _sc_gather(jnp.ravel(offset).astype(jnp.int32), weight)
    return out.reshape(B, T, D)
